# baseline (speedup 1.0000x reference)
import functools
import jax
import jax.numpy as jnp
from jax.experimental import pallas as pl
from jax.experimental.pallas import tpu as pltpu

N = 10000
E = 160000
NODE_DIM = 128
HIDDEN = 64
OUT = 64
HEADS = 8
HD = HEADS * HIDDEN

EB = 1024
E2 = E + N
NEB = (E2 + EB - 1) // EB
E2P = NEB * EB
NDB = (E + EB - 1) // EB
EDP = NDB * EB

NB = 1000
NNB = N // NB


def _embed_mm_kernel(xi_ref, emb_ref, W_ref, A_ref, h_ref, al_ref, g_ref):
    def body(j, _):
        s = xi_ref[0, 0, j]
        g_ref[pl.ds(j, 1), :] = emb_ref[pl.ds(s, 1), :]
        return 0
    jax.lax.fori_loop(0, NB, body, 0, unroll=16)
    h = jnp.dot(g_ref[...], W_ref[...], preferred_element_type=jnp.float32)
    h_ref[...] = h
    al_ref[...] = jnp.dot(h, A_ref[...], preferred_element_type=jnp.float32)


def _edge_kernel(src_ref, dst_ref, h_ref, al_ref,
                 nd_ref, g_ref, a_ref, v_ref, w_ref):
    gid = pl.program_id(0)

    @pl.when(gid == 0)
    def _():
        nd_ref[...] = jnp.zeros_like(nd_ref)

    def gather(j, _):
        s = src_ref[0, 0, j]
        d = dst_ref[0, 0, j]
        g_ref[pl.ds(j, 1), :] = h_ref[pl.ds(s, 1), :]
        a_ref[pl.ds(j, 1), :] = (al_ref[pl.ds(s, 1), :HEADS]
                                 + al_ref[pl.ds(d, 1), HEADS:])
        return 0
    jax.lax.fori_loop(0, EB, gather, 0, unroll=32)

    a = a_ref[...]
    e = jnp.where(a > 0, a, 0.2 * a)
    lim = E2 - gid * EB
    valid = jax.lax.broadcasted_iota(jnp.int32, (EB, 1), 0) < lim
    w = jnp.where(valid, jnp.exp(e), 0.0)
    w_ref[...] = w
    for k in range(HEADS):
        v_ref[:, k * HIDDEN:(k + 1) * HIDDEN] = (
            g_ref[:, k * HIDDEN:(k + 1) * HIDDEN]
            * jnp.broadcast_to(w[:, k:k + 1], (EB, HIDDEN)))

    v_ref[:, HD:] = w
    def scatter(j, _):
        d = dst_ref[0, 0, j]
        nd_ref[pl.ds(d, 1), :] = (nd_ref[pl.ds(d, 1), :]
                                  + v_ref[pl.ds(j, 1), :])
        return 0
    jax.lax.fori_loop(0, EB, scatter, 0, unroll=16)


def _finalize_mm_kernel(nd_ref, b_ref, W_ref, A_ref,
                        h_ref, al_ref):
    num = nd_ref[:, :HD]
    acc = jnp.zeros((NB, HIDDEN), dtype=jnp.float32)
    den = nd_ref[:, HD:]
    for k in range(HEADS):
        dk = jnp.broadcast_to(den[:, k:k + 1], (NB, HIDDEN)) + 1e-16
        acc = acc + num[:, k * HIDDEN:(k + 1) * HIDDEN] / dk
    hmid = acc * (1.0 / HEADS) + b_ref[...]
    hmid = jnp.maximum(hmid, 0.0)
    h = jnp.dot(hmid, W_ref[...], preferred_element_type=jnp.float32)
    h_ref[...] = h
    al_ref[...] = jnp.dot(h, A_ref[...], preferred_element_type=jnp.float32)


def _finalize_kernel(nd_ref, b_ref, z_ref):
    num = nd_ref[:, :HD]
    den = nd_ref[:, HD:]
    acc = jnp.zeros((NB, OUT), dtype=jnp.float32)
    for k in range(HEADS):
        dk = jnp.broadcast_to(den[:, k:k + 1], (NB, OUT)) + 1e-16
        acc = acc + num[:, k * OUT:(k + 1) * OUT] / dk
    z_ref[...] = acc * (1.0 / HEADS) + b_ref[...]


def _decode_kernel(si_ref, di_ref, z_ref, o_ref, s_ref, d_ref):
    def gather(j, _):
        s = si_ref[0, 0, j]
        d = di_ref[0, 0, j]
        s_ref[pl.ds(j, 1), :] = z_ref[pl.ds(s, 1), :]
        d_ref[pl.ds(j, 1), :] = z_ref[pl.ds(d, 1), :]
        return 0
    jax.lax.fori_loop(0, EB, gather, 0, unroll=32)
    r = jnp.sum(s_ref[...] * d_ref[...], axis=1)
    o_ref[...] = r.reshape(1, 8, 128)


def _pad_idx(a, tot):
    a = a.astype(jnp.int32)
    a = jnp.concatenate([a, jnp.zeros((tot - a.shape[0],), jnp.int32)])
    return a.reshape(-1, 1, EB)


def _blockdiag(a):
    h, d = a.shape
    return (a[:, :, None] * jnp.eye(h, dtype=a.dtype)[:, None, :]).reshape(h * d, h)


@jax.jit
def kernel(x, edge_index, edge_label_index, emb, W1, a1_src, a1_dst, b1,
           W2, a2_src, a2_dst, b2):
    loops = jnp.arange(N, dtype=jnp.int32)
    src = _pad_idx(jnp.concatenate([edge_index[0].astype(jnp.int32), loops]), E2P)
    dst = _pad_idx(jnp.concatenate([edge_index[1].astype(jnp.int32), loops]), E2P)
    xi = x[:, 0].astype(jnp.int32).reshape(NNB, 1, NB)
    A1 = jnp.concatenate([_blockdiag(a1_src), _blockdiag(a1_dst)], axis=1)
    A2 = jnp.concatenate([_blockdiag(a2_src), _blockdiag(a2_dst)], axis=1)

    f32 = jnp.float32
    smem_idx = pl.BlockSpec((1, 1, NB), lambda g: (g, 0, 0),
                            memory_space=pltpu.SMEM)
    smem_eidx = pl.BlockSpec((1, 1, EB), lambda g: (g, 0, 0),
                             memory_space=pltpu.SMEM)
    full = lambda shape: pl.BlockSpec(shape, lambda g: tuple(0 for _ in shape))
    nodeblk = lambda w: pl.BlockSpec((NB, w), lambda g: (g, 0))

    h1, al1 = pl.pallas_call(
        _embed_mm_kernel,
        grid=(NNB,),
        in_specs=[smem_idx, full((N, NODE_DIM)), full((NODE_DIM, HD)),
                  full((HD, 2 * HEADS))],
        out_specs=[nodeblk(HD), nodeblk(2 * HEADS)],
        out_shape=[jax.ShapeDtypeStruct((N, HD), f32),
                   jax.ShapeDtypeStruct((N, 2 * HEADS), f32)],
        scratch_shapes=[pltpu.VMEM((NB, NODE_DIM), f32)],
    )(xi, emb, W1, A1)

    def edge_pass(h, al):
        return pl.pallas_call(
            _edge_kernel,
            grid=(NEB,),
            in_specs=[smem_eidx, smem_eidx, full((N, HD)),
                      full((N, 2 * HEADS))],
            out_specs=full((N, HD + HEADS)),
            out_shape=jax.ShapeDtypeStruct((N, HD + HEADS), f32),
            scratch_shapes=[pltpu.VMEM((EB, HD), f32),
                            pltpu.VMEM((EB, HEADS), f32),
                            pltpu.VMEM((EB, HD + HEADS), f32),
                            pltpu.VMEM((EB, HEADS), f32)],
        )(src, dst, h, al)

    nd1 = edge_pass(h1, al1)

    h2, al2 = pl.pallas_call(
        _finalize_mm_kernel,
        grid=(NNB,),
        in_specs=[nodeblk(HD + HEADS), full((1, HIDDEN)),
                  full((HIDDEN, HD)), full((HD, 2 * HEADS))],
        out_specs=[nodeblk(HD), nodeblk(2 * HEADS)],
        out_shape=[jax.ShapeDtypeStruct((N, HD), f32),
                   jax.ShapeDtypeStruct((N, 2 * HEADS), f32)],
    )(nd1, b1.reshape(1, HIDDEN), W2, A2)

    nd2 = edge_pass(h2, al2)

    z = pl.pallas_call(
        _finalize_kernel,
        grid=(NNB,),
        in_specs=[nodeblk(HD + HEADS), full((1, OUT))],
        out_specs=nodeblk(OUT),
        out_shape=jax.ShapeDtypeStruct((N, OUT), f32),
    )(nd2, b2.reshape(1, OUT))

    si = _pad_idx(edge_label_index[0], EDP)
    di = _pad_idx(edge_label_index[1], EDP)
    out = pl.pallas_call(
        _decode_kernel,
        grid=(NDB,),
        in_specs=[smem_eidx, smem_eidx, full((N, OUT))],
        out_specs=pl.BlockSpec((1, 8, 128), lambda g: (g, 0, 0)),
        out_shape=jax.ShapeDtypeStruct((NDB, 8, 128), f32),
        scratch_shapes=[pltpu.VMEM((EB, OUT), f32),
                        pltpu.VMEM((EB, OUT), f32)],
    )(si, di, z)
    return out.reshape(EDP)[:E]


# speedup vs baseline: 1.0667x; 1.0667x over previous
import functools
import jax
import jax.numpy as jnp
from jax.experimental import pallas as pl
from jax.experimental.pallas import tpu as pltpu

N = 10000
E = 160000
NODE_DIM = 128
HIDDEN = 64
OUT = 64
HEADS = 8
HD = HEADS * HIDDEN

EB = 1024
E2 = E + N
NEB = (E2 + EB - 1) // EB
E2P = NEB * EB
NDB = (E + EB - 1) // EB
EDP = NDB * EB

NB = 1000
NNB = N // NB


def _embed_mm_kernel(xi_ref, emb_ref, W_ref, A_ref, h_ref, al_ref, g_ref):
    def body(j, _):
        s = xi_ref[0, 0, j]
        g_ref[pl.ds(j, 1), :] = emb_ref[pl.ds(s, 1), :]
        return 0
    jax.lax.fori_loop(0, NB, body, 0, unroll=16)
    h = jnp.dot(g_ref[...], W_ref[...], preferred_element_type=jnp.float32)
    h_ref[...] = h
    al_ref[...] = jnp.dot(h, A_ref[...], preferred_element_type=jnp.float32)


def _edge_kernel(src_ref, dst_ref, h_ref, al_ref,
                 num_ref, den_ref, g_ref, a_ref, v_ref, w_ref):
    gid = pl.program_id(0)

    @pl.when(gid == 0)
    def _():
        num_ref[...] = jnp.zeros_like(num_ref)
        den_ref[...] = jnp.zeros_like(den_ref)

    def gather(j, _):
        s = src_ref[0, 0, j]
        d = dst_ref[0, 0, j]
        g_ref[pl.ds(j, 1), :] = h_ref[pl.ds(s, 1), :]
        a_ref[pl.ds(j, 1), :] = (al_ref[pl.ds(s, 1), :HEADS]
                                 + al_ref[pl.ds(d, 1), HEADS:])
        return 0
    jax.lax.fori_loop(0, EB, gather, 0, unroll=32)

    a = a_ref[...]
    e = jnp.where(a > 0, a, 0.2 * a)
    lim = E2 - gid * EB
    valid = jax.lax.broadcasted_iota(jnp.int32, (EB, 1), 0) < lim
    w = jnp.where(valid, jnp.exp(e), 0.0)
    w_ref[...] = w
    for k in range(HEADS):
        v_ref[:, k * HIDDEN:(k + 1) * HIDDEN] = (
            g_ref[:, k * HIDDEN:(k + 1) * HIDDEN]
            * jnp.broadcast_to(w[:, k:k + 1], (EB, HIDDEN)))

    def scatter(j, _):
        d = dst_ref[0, 0, j]
        num_ref[pl.ds(d, 1), :] = (num_ref[pl.ds(d, 1), :]
                                   + v_ref[pl.ds(j, 1), :])
        den_ref[pl.ds(d, 1), :] = (den_ref[pl.ds(d, 1), :]
                                   + w_ref[pl.ds(j, 1), :])
        return 0
    jax.lax.fori_loop(0, EB, scatter, 0, unroll=16)


def _finalize_mm_kernel(num_ref, den_ref, b_ref, W_ref, A_ref,
                        h_ref, al_ref):
    num = num_ref[...]
    acc = jnp.zeros((NB, HIDDEN), dtype=jnp.float32)
    den = den_ref[...]
    for k in range(HEADS):
        dk = jnp.broadcast_to(den[:, k:k + 1], (NB, HIDDEN)) + 1e-16
        acc = acc + num[:, k * HIDDEN:(k + 1) * HIDDEN] / dk
    hmid = acc * (1.0 / HEADS) + b_ref[...]
    hmid = jnp.maximum(hmid, 0.0)
    h = jnp.dot(hmid, W_ref[...], preferred_element_type=jnp.float32)
    h_ref[...] = h
    al_ref[...] = jnp.dot(h, A_ref[...], preferred_element_type=jnp.float32)


def _finalize_kernel(num_ref, den_ref, b_ref, z_ref):
    num = num_ref[...]
    den = den_ref[...]
    acc = jnp.zeros((NB, OUT), dtype=jnp.float32)
    for k in range(HEADS):
        dk = jnp.broadcast_to(den[:, k:k + 1], (NB, OUT)) + 1e-16
        acc = acc + num[:, k * OUT:(k + 1) * OUT] / dk
    z_ref[...] = acc * (1.0 / HEADS) + b_ref[...]


def _decode_kernel(si_ref, di_ref, z_ref, o_ref, s_ref, d_ref):
    def gather(j, _):
        s = si_ref[0, 0, j]
        d = di_ref[0, 0, j]
        s_ref[pl.ds(j, 1), :] = z_ref[pl.ds(s, 1), :]
        d_ref[pl.ds(j, 1), :] = z_ref[pl.ds(d, 1), :]
        return 0
    jax.lax.fori_loop(0, EB, gather, 0, unroll=32)
    r = jnp.sum(s_ref[...] * d_ref[...], axis=1)
    o_ref[...] = r.reshape(1, 8, 128)


def _pad_idx(a, tot):
    a = a.astype(jnp.int32)
    a = jnp.concatenate([a, jnp.zeros((tot - a.shape[0],), jnp.int32)])
    return a.reshape(-1, 1, EB)


def _blockdiag(a):
    h, d = a.shape
    return (a[:, :, None] * jnp.eye(h, dtype=a.dtype)[:, None, :]).reshape(h * d, h)


@jax.jit
def kernel(x, edge_index, edge_label_index, emb, W1, a1_src, a1_dst, b1,
           W2, a2_src, a2_dst, b2):
    loops = jnp.arange(N, dtype=jnp.int32)
    src = _pad_idx(jnp.concatenate([edge_index[0].astype(jnp.int32), loops]), E2P)
    dst = _pad_idx(jnp.concatenate([edge_index[1].astype(jnp.int32), loops]), E2P)
    xi = x[:, 0].astype(jnp.int32).reshape(NNB, 1, NB)
    A1 = jnp.concatenate([_blockdiag(a1_src), _blockdiag(a1_dst)], axis=1)
    A2 = jnp.concatenate([_blockdiag(a2_src), _blockdiag(a2_dst)], axis=1)

    f32 = jnp.float32
    smem_idx = pl.BlockSpec((1, 1, NB), lambda g: (g, 0, 0),
                            memory_space=pltpu.SMEM)
    smem_eidx = pl.BlockSpec((1, 1, EB), lambda g: (g, 0, 0),
                             memory_space=pltpu.SMEM)
    full = lambda shape: pl.BlockSpec(shape, lambda g: tuple(0 for _ in shape))
    nodeblk = lambda w: pl.BlockSpec((NB, w), lambda g: (g, 0))

    h1, al1 = pl.pallas_call(
        _embed_mm_kernel,
        grid=(NNB,),
        in_specs=[smem_idx, full((N, NODE_DIM)), full((NODE_DIM, HD)),
                  full((HD, 2 * HEADS))],
        out_specs=[nodeblk(HD), nodeblk(2 * HEADS)],
        out_shape=[jax.ShapeDtypeStruct((N, HD), f32),
                   jax.ShapeDtypeStruct((N, 2 * HEADS), f32)],
        scratch_shapes=[pltpu.VMEM((NB, NODE_DIM), f32)],
    )(xi, emb, W1, A1)

    def edge_pass(h, al):
        return pl.pallas_call(
            _edge_kernel,
            grid=(NEB,),
            in_specs=[smem_eidx, smem_eidx, full((N, HD)),
                      full((N, 2 * HEADS))],
            out_specs=[full((N, HD)), full((N, HEADS))],
            out_shape=[jax.ShapeDtypeStruct((N, HD), f32),
                       jax.ShapeDtypeStruct((N, HEADS), f32)],
            scratch_shapes=[pltpu.VMEM((EB, HD), f32),
                            pltpu.VMEM((EB, HEADS), f32),
                            pltpu.VMEM((EB, HD), f32),
                            pltpu.VMEM((EB, HEADS), f32)],
        )(src, dst, h, al)

    num1, den1 = edge_pass(h1, al1)

    h2, al2 = pl.pallas_call(
        _finalize_mm_kernel,
        grid=(NNB,),
        in_specs=[nodeblk(HD), nodeblk(HEADS), full((1, HIDDEN)),
                  full((HIDDEN, HD)), full((HD, 2 * HEADS))],
        out_specs=[nodeblk(HD), nodeblk(2 * HEADS)],
        out_shape=[jax.ShapeDtypeStruct((N, HD), f32),
                   jax.ShapeDtypeStruct((N, 2 * HEADS), f32)],
    )(num1, den1, b1.reshape(1, HIDDEN), W2, A2)

    num2, den2 = edge_pass(h2, al2)

    z = pl.pallas_call(
        _finalize_kernel,
        grid=(NNB,),
        in_specs=[nodeblk(HD), nodeblk(HEADS), full((1, OUT))],
        out_specs=nodeblk(OUT),
        out_shape=jax.ShapeDtypeStruct((N, OUT), f32),
    )(num2, den2, b2.reshape(1, OUT))

    si = _pad_idx(edge_label_index[0], EDP)
    di = _pad_idx(edge_label_index[1], EDP)
    out = pl.pallas_call(
        _decode_kernel,
        grid=(NDB,),
        in_specs=[smem_eidx, smem_eidx, full((N, OUT))],
        out_specs=pl.BlockSpec((1, 8, 128), lambda g: (g, 0, 0)),
        out_shape=jax.ShapeDtypeStruct((NDB, 8, 128), f32),
        scratch_shapes=[pltpu.VMEM((EB, OUT), f32),
                        pltpu.VMEM((EB, OUT), f32)],
    )(si, di, z)
    return out.reshape(EDP)[:E]
